# baseline (speedup 1.0000x reference)
import functools

import jax
import jax.numpy as jnp
from jax import lax
from jax.experimental import pallas as pl
from jax.experimental.pallas import tpu as pltpu
from jax.experimental.pallas import tpu_sc as plsc

EMBED_DIM = 64
LANES = 16
NUM_CORES = 2
NUM_SUBCORES = 16
NUM_WORKERS = NUM_CORES * NUM_SUBCORES
IDX_CHUNK = 128


def _rsqrt_newton(x):
    i = lax.bitcast_convert_type(x, jnp.int32)
    i = jnp.int32(0x5F3759DF) - lax.shift_right_logical(i, 1)
    y = lax.bitcast_convert_type(i, jnp.float32)
    for _ in range(3):
        y = y * (jnp.float32(1.5) - jnp.float32(0.5) * x * y * y)
    return y


def kernel(nodes, table):
    batch = nodes.shape[0]
    b_per_w = batch // NUM_WORKERS
    n_chunks = b_per_w // IDX_CHUNK
    nodes2d = nodes.reshape(NUM_WORKERS * n_chunks, IDX_CHUNK)

    mesh = plsc.VectorSubcoreMesh(core_axis_name="c", subcore_axis_name="s")

    @functools.partial(
        pl.kernel,
        mesh=mesh,
        out_type=jax.ShapeDtypeStruct((batch, EMBED_DIM), jnp.float32),
        scratch_types=[
            pltpu.VMEM((n_chunks, IDX_CHUNK), jnp.int32),
            pltpu.VMEM((b_per_w, EMBED_DIM), jnp.float32),
            pltpu.SemaphoreType.DMA,
        ],
        compiler_params=pltpu.CompilerParams(
            needs_layout_passes=False, use_tc_tiling_on_sc=False
        ),
    )
    def sc_kernel(nodes_hbm, table_hbm, out_hbm, idx_v, rows_v, sem):
        wid = lax.axis_index("s") * NUM_CORES + lax.axis_index("c")
        base = wid * b_per_w
        pltpu.sync_copy(nodes_hbm.at[pl.ds(wid * n_chunks, n_chunks)], idx_v)
        copies = [
            pltpu.async_copy(
                table_hbm.at[idx_v.at[j]],
                rows_v.at[pl.ds(j * IDX_CHUNK, IDX_CHUNK)],
                sem,
            )
            for j in range(n_chunks)
        ]
        for c in copies:
            c.wait()

        lane = lax.iota(jnp.int32, LANES)

        def group_body(g, carry):
            row_ids = g * LANES + lane
            acc = jnp.zeros((LANES,), jnp.float32)
            for j in range(EMBED_DIM):
                col = plsc.load_gather(
                    rows_v, [row_ids, jnp.full((LANES,), j, jnp.int32)]
                )
                acc = acc + col * col
            inv = _rsqrt_newton(jnp.maximum(acc, jnp.float32(1e-24)))
            for i in range(LANES):
                r = g * LANES + i
                s = inv[i]
                for k in range(EMBED_DIM // LANES):
                    sl = pl.ds(k * LANES, LANES)
                    rows_v[r, sl] = rows_v[r, sl] * s
            return carry

        lax.fori_loop(0, b_per_w // LANES, group_body, 0)
        pltpu.sync_copy(rows_v, out_hbm.at[pl.ds(base, b_per_w)])

    return sc_kernel(nodes2d, table)


# speedup vs baseline: 3.1717x; 3.1717x over previous
import functools

import jax
import jax.numpy as jnp
from jax import lax
from jax.experimental import pallas as pl
from jax.experimental.pallas import tpu as pltpu
from jax.experimental.pallas import tpu_sc as plsc

EMBED_DIM = 64
LANES = 16
NUM_CORES = 2
NUM_SUBCORES = 16
NUM_WORKERS = NUM_CORES * NUM_SUBCORES
WIN = 128
RING = 8


def _rsqrt_newton(x):
    i = lax.bitcast_convert_type(x, jnp.int32)
    i = jnp.int32(0x5F3759DF) - lax.shift_right_logical(i, 1)
    y = lax.bitcast_convert_type(i, jnp.float32)
    for _ in range(3):
        y = y * (jnp.float32(1.5) - jnp.float32(0.5) * x * y * y)
    return y


def kernel(nodes, table):
    batch = nodes.shape[0]
    n_nodes = table.shape[0]
    b_per_w = batch // NUM_WORKERS
    table_t = table.T
    tail_lo = (n_nodes // WIN) * WIN
    tail_n = n_nodes - tail_lo
    tail_t = table_t[:, tail_lo:]

    mesh = plsc.VectorSubcoreMesh(core_axis_name="c", subcore_axis_name="s")

    @functools.partial(
        pl.kernel,
        mesh=mesh,
        out_type=jax.ShapeDtypeStruct((EMBED_DIM, batch), jnp.float32),
        scratch_types=[
            pltpu.VMEM((b_per_w + 2 * LANES,), jnp.int32),
            pltpu.VMEM((RING, EMBED_DIM, WIN), jnp.float32),
            pltpu.VMEM((EMBED_DIM, tail_n), jnp.float32),
            pltpu.VMEM((EMBED_DIM, b_per_w), jnp.float32),
            pltpu.SemaphoreType.DMA,
            pltpu.SemaphoreType.DMA,
        ],
        compiler_params=pltpu.CompilerParams(
            needs_layout_passes=False, use_tc_tiling_on_sc=True
        ),
    )
    def sc_kernel(
        nodes_hbm, table_hbm, tail_hbm, out_hbm, idx_vm, win_v, tail_v,
        cols_v, sem, ring_sem,
    ):
        wid = lax.axis_index("s") * NUM_CORES + lax.axis_index("c")
        base = wid * b_per_w
        pltpu.sync_copy(nodes_hbm.at[pl.ds(base, b_per_w)],
                        idx_vm.at[pl.ds(0, b_per_w)])
        pltpu.sync_copy(tail_hbm, tail_v)

        lane = lax.iota(jnp.int32, LANES)

        def win_copy(n, slot):
            nc = jnp.minimum(n, jnp.int32(tail_lo - 1))
            w0 = (nc // WIN) * WIN
            return pltpu.async_copy(
                table_hbm.at[:, pl.ds(w0, WIN)], win_v.at[slot], ring_sem
            )

        head = idx_vm[pl.ds(0, LANES)]
        for j in range(RING):
            win_copy(head[j], j)

        def pick_body(g, carry):
            vec = idx_vm[pl.ds(g * RING, 2 * RING)]
            for j in range(RING):
                n = vec[j]
                i = g * RING + j
                pltpu.make_async_copy(
                    table_hbm.at[:, pl.ds(0, WIN)], win_v.at[j], ring_sem
                ).wait()
                col = lax.broadcast(lax.rem(n, jnp.int32(WIN)), (LANES,))
                ivec = lax.broadcast(i, (LANES,))
                for k in range(EMBED_DIM // LANES):
                    d_idx = lane + k * LANES
                    v = plsc.load_gather(win_v.at[j], [d_idx, col])
                    plsc.store_scatter(cols_v, [d_idx, ivec], v)

                @pl.when(n >= tail_lo)
                def _():
                    tcol = lax.broadcast(n - jnp.int32(tail_lo), (LANES,))
                    for k in range(EMBED_DIM // LANES):
                        d_idx = lane + k * LANES
                        v = plsc.load_gather(tail_v, [d_idx, tcol])
                        plsc.store_scatter(cols_v, [d_idx, ivec], v)

                @pl.when(g < b_per_w // RING - 1)
                def _():
                    win_copy(vec[RING + j], j)

            return carry

        lax.fori_loop(0, b_per_w // RING, pick_body, 0)

        def group_body(g, carry):
            sl = pl.ds(g * LANES, LANES)
            acc = jnp.zeros((LANES,), jnp.float32)
            for d in range(EMBED_DIM):
                v = cols_v[d, sl]
                acc = acc + v * v
            inv = _rsqrt_newton(jnp.maximum(acc, jnp.float32(1e-24)))
            for d in range(EMBED_DIM):
                cols_v[d, sl] = cols_v[d, sl] * inv
            return carry

        lax.fori_loop(0, b_per_w // LANES, group_body, 0)
        pltpu.sync_copy(cols_v, out_hbm.at[:, pl.ds(base, b_per_w)])

    out_t = sc_kernel(nodes, table_t, tail_t)
    return out_t.T
